# baseline (speedup 1.0000x reference)
import functools

import jax
import jax.numpy as jnp
from jax import lax
from jax.experimental import pallas as pl
from jax.experimental.pallas import tpu as pltpu
from jax.experimental.pallas import tpu_sc as plsc

NUM_USERS = 1000000
NUM_MOVIES = 100000
EMBED_DIM = 64
BATCH = 16384
NH = 8

NC = 2
NS = 16
NW = NC * NS
B_PER_W = BATCH // NW
N_CHUNK = 4
CHUNK = B_PER_W // N_CHUNK
D2 = 128
UPR = D2 // NH
SEGB = 512
UPB = SEGB * UPR


def _proj_body(x_ref, w_ref, o_ref):
    o_ref[...] = lax.dot_general(
        w_ref[...], x_ref[...], (((0,), (0,)), ((), ())),
        preferred_element_type=jnp.float32)


def _proj_stage(xt, w, n):
    nb = -(-n // UPB)
    return pl.pallas_call(
        _proj_body,
        grid=(nb,),
        in_specs=[
            pl.BlockSpec((EMBED_DIM, UPB), lambda b: (0, b)),
            pl.BlockSpec((EMBED_DIM, NH), lambda b: (0, 0)),
        ],
        out_specs=pl.BlockSpec((NH, UPB), lambda b: (0, b)),
        out_shape=jax.ShapeDtypeStruct((NH, nb * UPB), jnp.float32),
        compiler_params=pltpu.CompilerParams(
            dimension_semantics=("parallel",)),
    )(xt, w)


def _sc_gather(uidx2d, midx2d, zu, zm):
    mesh = plsc.VectorSubcoreMesh(core_axis_name="c", subcore_axis_name="s")
    out_sds = jax.ShapeDtypeStruct((NW * N_CHUNK, CHUNK, D2), jnp.float32)

    @functools.partial(
        pl.kernel,
        out_type=(out_sds, out_sds),
        mesh=mesh,
        scratch_types=[
            pltpu.VMEM((N_CHUNK, CHUNK), jnp.int32),
            pltpu.VMEM((N_CHUNK, CHUNK), jnp.int32),
            pltpu.VMEM((N_CHUNK, CHUNK, D2), jnp.float32),
            pltpu.VMEM((N_CHUNK - 1, CHUNK, D2), jnp.float32),
            pltpu.SemaphoreType.DMA,
            pltpu.SemaphoreType.DMA,
            pltpu.SemaphoreType.DMA,
        ],
    )
    def k(uid_hbm, mid_hbm, zu_hbm, zm_hbm, u_out, m_out, idxu_v, idxm_v,
          bufu_v, bufm_v, sem_g, sem_wu, sem_wm):
        wid = lax.axis_index("s") * NC + lax.axis_index("c")
        base = wid * N_CHUNK
        pltpu.sync_copy(uid_hbm.at[pl.ds(base, N_CHUNK)], idxu_v)
        pltpu.sync_copy(mid_hbm.at[pl.ds(base, N_CHUNK)], idxm_v)
        gu = [pltpu.async_copy(zu_hbm.at[idxu_v.at[j]], bufu_v.at[j], sem_g)
              for j in range(N_CHUNK)]
        gm = [pltpu.async_copy(zm_hbm.at[idxm_v.at[j]], bufm_v.at[j], sem_g)
              for j in range(N_CHUNK - 1)]
        for c in gu:
            c.wait()
        wu = pltpu.async_copy(bufu_v, u_out.at[pl.ds(base, N_CHUNK)], sem_wu)
        for c in gm:
            c.wait()
        wm0 = pltpu.async_copy(
            bufm_v, m_out.at[pl.ds(base, N_CHUNK - 1)], sem_wm)
        wu.wait()
        glast = pltpu.async_copy(
            zm_hbm.at[idxm_v.at[N_CHUNK - 1]], bufu_v.at[0], sem_g)
        glast.wait()
        wm1 = pltpu.async_copy(
            bufu_v.at[0], m_out.at[base + N_CHUNK - 1], sem_wm)
        wm0.wait()
        wm1.wait()

    return k(uidx2d, midx2d, zu, zm)


BR = 2048


def _mlp_body(gu_ref, gm_ref, pu_ref, pm_ref, summ_ref, b1_ref, w2_ref,
              b2_ref, o_ref):
    grp = jax.lax.broadcasted_iota(jnp.int32, (BR, D2), 1) >> 3
    sel = (jnp.where(grp == pu_ref[...], gu_ref[...], 0.0)
           + jnp.where(grp == pm_ref[...], gm_ref[...], 0.0))
    x = jnp.dot(sel, summ_ref[...], preferred_element_type=jnp.float32)
    h = jnp.maximum(x + b1_ref[...], 0.0)
    o = jnp.dot(h, w2_ref[...], preferred_element_type=jnp.float32) + b2_ref[...]
    o_ref[...] = jax.nn.sigmoid(o) * 4.0 + 1.0


def _tc_mlp(gu, gm, pu, pm, summ, b1, W2, b2):
    grid = (BATCH // BR,)
    return pl.pallas_call(
        _mlp_body,
        grid=grid,
        in_specs=[
            pl.BlockSpec((BR, D2), lambda i: (i, 0)),
            pl.BlockSpec((BR, D2), lambda i: (i, 0)),
            pl.BlockSpec((BR, 1), lambda i: (i, 0)),
            pl.BlockSpec((BR, 1), lambda i: (i, 0)),
            pl.BlockSpec((D2, NH), lambda i: (0, 0)),
            pl.BlockSpec((1, NH), lambda i: (0, 0)),
            pl.BlockSpec((NH, 1), lambda i: (0, 0)),
            pl.BlockSpec((1, 1), lambda i: (0, 0)),
        ],
        out_specs=pl.BlockSpec((BR, 1), lambda i: (i, 0)),
        out_shape=jax.ShapeDtypeStruct((BATCH, 1), jnp.float32),
    )(gu, gm, pu, pm, summ, b1, W2, b2)


def kernel(user_ids, movie_ids, user_table, movie_table, W1, b1, W2, b2):
    uid = user_ids.astype(jnp.int32)
    mid = movie_ids.astype(jnp.int32)
    pu_fm = _proj_stage(user_table.T, W1[:EMBED_DIM], NUM_USERS)
    pm_fm = _proj_stage(movie_table.T, W1[EMBED_DIM:], NUM_MOVIES)
    zu = pu_fm[:, :NUM_USERS].T.reshape(NUM_USERS // UPR, D2)
    zm = pm_fm[:, :NUM_MOVIES].T.reshape(NUM_MOVIES // UPR, D2)
    gu3, gm3 = _sc_gather((uid >> 4).reshape(NW * N_CHUNK, CHUNK),
                          (mid >> 4).reshape(NW * N_CHUNK, CHUNK), zu, zm)
    gu = gu3.reshape(BATCH, D2)
    gm = gm3.reshape(BATCH, D2)
    summ = jnp.tile(jnp.eye(NH, dtype=jnp.float32), (UPR, 1))
    out = _tc_mlp(gu, gm, (uid & 15).reshape(BATCH, 1),
                  (mid & 15).reshape(BATCH, 1), summ, b1.reshape(1, NH), W2,
                  b2.reshape(1, 1))
    return out.reshape(BATCH)


# speedup vs baseline: 3.0177x; 3.0177x over previous
import functools

import jax
import jax.numpy as jnp
from jax import lax
from jax.experimental import pallas as pl
from jax.experimental.pallas import tpu as pltpu
from jax.experimental.pallas import tpu_sc as plsc

NUM_USERS = 1000000
NUM_MOVIES = 100000
EMBED_DIM = 64
BATCH = 16384
NH = 8

NC = 2
NS = 16
NW = NC * NS
B_PER_W = BATCH // NW
N_CHUNK = 4
CHUNK = B_PER_W // N_CHUNK
D2 = 128
UPR = D2 // NH
SEGF = 512
UPB = SEGF * UPR


def _proj_body(x_ref, w_ref, eye_ref, o_ref, *, n):
    dn = (((0,), (0,)), ((), ()))
    p = lax.dot_general(w_ref[...], x_ref[...], dn,
                        preferred_element_type=jnp.float32)
    nvalid = jnp.int32(n) - pl.program_id(0) * UPB
    lane = jax.lax.broadcasted_iota(jnp.int32, (NH, UPB), 1)
    p = jnp.where(lane < nvalid, p, 0.0)
    p2 = jnp.concatenate([p[:, g * SEGF:(g + 1) * SEGF] for g in range(UPR)],
                         axis=0)
    o_ref[...] = lax.dot_general(p2, eye_ref[...], dn,
                                 preferred_element_type=jnp.float32)


def _proj_stage(xt, w, eye128, n):
    nb = -(-n // UPB)
    return pl.pallas_call(
        functools.partial(_proj_body, n=n),
        grid=(nb,),
        in_specs=[
            pl.BlockSpec((EMBED_DIM, UPB), lambda b: (0, b)),
            pl.BlockSpec((EMBED_DIM, NH), lambda b: (0, 0)),
            pl.BlockSpec((D2, D2), lambda b: (0, 0)),
        ],
        out_specs=pl.BlockSpec((SEGF, D2), lambda b: (b, 0)),
        out_shape=jax.ShapeDtypeStruct((nb * SEGF, D2), jnp.float32),
        compiler_params=pltpu.CompilerParams(
            dimension_semantics=("parallel",)),
    )(xt, w, eye128)


def _sc_gather(uidx2d, midx2d, zu, zm):
    mesh = plsc.VectorSubcoreMesh(core_axis_name="c", subcore_axis_name="s")
    out_sds = jax.ShapeDtypeStruct((NW * N_CHUNK, CHUNK, D2), jnp.float32)

    @functools.partial(
        pl.kernel,
        out_type=(out_sds, out_sds),
        mesh=mesh,
        scratch_types=[
            pltpu.VMEM((N_CHUNK, CHUNK), jnp.int32),
            pltpu.VMEM((N_CHUNK, CHUNK), jnp.int32),
            pltpu.VMEM((N_CHUNK, CHUNK, D2), jnp.float32),
            pltpu.VMEM((N_CHUNK - 1, CHUNK, D2), jnp.float32),
            pltpu.SemaphoreType.DMA,
            pltpu.SemaphoreType.DMA,
            pltpu.SemaphoreType.DMA,
        ],
    )
    def k(uid_hbm, mid_hbm, zu_hbm, zm_hbm, u_out, m_out, idxu_v, idxm_v,
          bufu_v, bufm_v, sem_g, sem_wu, sem_wm):
        wid = lax.axis_index("s") * NC + lax.axis_index("c")
        base = wid * N_CHUNK
        pltpu.sync_copy(uid_hbm.at[pl.ds(base, N_CHUNK)], idxu_v)
        pltpu.sync_copy(mid_hbm.at[pl.ds(base, N_CHUNK)], idxm_v)
        gu = [pltpu.async_copy(zu_hbm.at[idxu_v.at[j]], bufu_v.at[j], sem_g)
              for j in range(N_CHUNK)]
        gm = [pltpu.async_copy(zm_hbm.at[idxm_v.at[j]], bufm_v.at[j], sem_g)
              for j in range(N_CHUNK - 1)]
        for c in gu:
            c.wait()
        wu = pltpu.async_copy(bufu_v, u_out.at[pl.ds(base, N_CHUNK)], sem_wu)
        for c in gm:
            c.wait()
        wm0 = pltpu.async_copy(
            bufm_v, m_out.at[pl.ds(base, N_CHUNK - 1)], sem_wm)
        wu.wait()
        glast = pltpu.async_copy(
            zm_hbm.at[idxm_v.at[N_CHUNK - 1]], bufu_v.at[0], sem_g)
        glast.wait()
        wm1 = pltpu.async_copy(
            bufu_v.at[0], m_out.at[base + N_CHUNK - 1], sem_wm)
        wm0.wait()
        wm1.wait()

    return k(uidx2d, midx2d, zu, zm)


BR = 2048


def _mlp_body(gu_ref, gm_ref, pu_ref, pm_ref, summ_ref, b1_ref, w2_ref,
              b2_ref, o_ref):
    grp = jax.lax.broadcasted_iota(jnp.int32, (BR, D2), 1) >> 3
    sel = (jnp.where(grp == pu_ref[...], gu_ref[...], 0.0)
           + jnp.where(grp == pm_ref[...], gm_ref[...], 0.0))
    x = jnp.dot(sel, summ_ref[...], preferred_element_type=jnp.float32)
    h = jnp.maximum(x + b1_ref[...], 0.0)
    o = jnp.dot(h, w2_ref[...], preferred_element_type=jnp.float32) + b2_ref[...]
    o_ref[...] = jax.nn.sigmoid(o) * 4.0 + 1.0


def _tc_mlp(gu, gm, pu, pm, summ, b1, W2, b2):
    grid = (BATCH // BR,)
    return pl.pallas_call(
        _mlp_body,
        grid=grid,
        in_specs=[
            pl.BlockSpec((BR, D2), lambda i: (i, 0)),
            pl.BlockSpec((BR, D2), lambda i: (i, 0)),
            pl.BlockSpec((BR, 1), lambda i: (i, 0)),
            pl.BlockSpec((BR, 1), lambda i: (i, 0)),
            pl.BlockSpec((D2, NH), lambda i: (0, 0)),
            pl.BlockSpec((1, NH), lambda i: (0, 0)),
            pl.BlockSpec((NH, 1), lambda i: (0, 0)),
            pl.BlockSpec((1, 1), lambda i: (0, 0)),
        ],
        out_specs=pl.BlockSpec((BR, 1), lambda i: (i, 0)),
        out_shape=jax.ShapeDtypeStruct((BATCH, 1), jnp.float32),
    )(gu, gm, pu, pm, summ, b1, W2, b2)


def kernel(user_ids, movie_ids, user_table, movie_table, W1, b1, W2, b2):
    uid = user_ids.astype(jnp.int32)
    mid = movie_ids.astype(jnp.int32)
    eye128 = jnp.eye(D2, dtype=jnp.float32)
    zu = _proj_stage(user_table.T, W1[:EMBED_DIM], eye128, NUM_USERS)
    zm = _proj_stage(movie_table.T, W1[EMBED_DIM:], eye128, NUM_MOVIES)
    urow = ((uid >> 13) << 9) | (uid & (SEGF - 1))
    mrow = ((mid >> 13) << 9) | (mid & (SEGF - 1))
    gu3, gm3 = _sc_gather(urow.reshape(NW * N_CHUNK, CHUNK),
                          mrow.reshape(NW * N_CHUNK, CHUNK), zu, zm)
    gu = gu3.reshape(BATCH, D2)
    gm = gm3.reshape(BATCH, D2)
    summ = jnp.tile(jnp.eye(NH, dtype=jnp.float32), (UPR, 1))
    out = _tc_mlp(gu, gm, ((uid >> 9) & 15).reshape(BATCH, 1),
                  ((mid >> 9) & 15).reshape(BATCH, 1), summ,
                  b1.reshape(1, NH), W2, b2.reshape(1, 1))
    return out.reshape(BATCH)


# speedup vs baseline: 3.8062x; 1.2613x over previous
import functools

import jax
import jax.numpy as jnp
from jax import lax
from jax.experimental import pallas as pl
from jax.experimental.pallas import tpu as pltpu
from jax.experimental.pallas import tpu_sc as plsc

NUM_USERS = 1000000
NUM_MOVIES = 100000
EMBED_DIM = 64
BATCH = 16384
NH = 8

NC = 2
NS = 16
NW = NC * NS
B_PER_W = BATCH // NW
N_CHUNK = 4
CHUNK = B_PER_W // N_CHUNK
D2 = 128
UPR = D2 // NH
SEGF = 1024
UPB = SEGF * UPR


def _proj_body(x_ref, w_ref, eye_ref, o_ref, *, n):
    dn = (((0,), (0,)), ((), ()))
    p = lax.dot_general(w_ref[...], x_ref[...], dn,
                        preferred_element_type=jnp.float32)
    nvalid = jnp.int32(n) - pl.program_id(0) * UPB
    lane = jax.lax.broadcasted_iota(jnp.int32, (NH, UPB), 1)
    p = jnp.where(lane < nvalid, p, 0.0)
    p2 = jnp.concatenate([p[:, g * SEGF:(g + 1) * SEGF] for g in range(UPR)],
                         axis=0)
    o_ref[...] = lax.dot_general(p2, eye_ref[...], dn,
                                 preferred_element_type=jnp.float32)


def _proj_stage(xt, w, eye128, n):
    nb = -(-n // UPB)
    return pl.pallas_call(
        functools.partial(_proj_body, n=n),
        grid=(nb,),
        in_specs=[
            pl.BlockSpec((EMBED_DIM, UPB), lambda b: (0, b)),
            pl.BlockSpec((EMBED_DIM, NH), lambda b: (0, 0)),
            pl.BlockSpec((D2, D2), lambda b: (0, 0)),
        ],
        out_specs=pl.BlockSpec((SEGF, D2), lambda b: (b, 0)),
        out_shape=jax.ShapeDtypeStruct((nb * SEGF, D2), jnp.float32),
        compiler_params=pltpu.CompilerParams(
            dimension_semantics=("parallel",)),
    )(xt, w, eye128)


def _sc_gather(uidx2d, midx2d, zu, zm):
    mesh = plsc.VectorSubcoreMesh(core_axis_name="c", subcore_axis_name="s")
    out_sds = jax.ShapeDtypeStruct((NW * N_CHUNK, CHUNK, D2), jnp.float32)

    @functools.partial(
        pl.kernel,
        out_type=(out_sds, out_sds),
        mesh=mesh,
        scratch_types=[
            pltpu.VMEM((N_CHUNK, CHUNK), jnp.int32),
            pltpu.VMEM((N_CHUNK, CHUNK), jnp.int32),
            pltpu.VMEM((N_CHUNK, CHUNK, D2), jnp.float32),
            pltpu.VMEM((N_CHUNK - 1, CHUNK, D2), jnp.float32),
            pltpu.SemaphoreType.DMA,
            pltpu.SemaphoreType.DMA,
            pltpu.SemaphoreType.DMA,
        ],
    )
    def k(uid_hbm, mid_hbm, zu_hbm, zm_hbm, u_out, m_out, idxu_v, idxm_v,
          bufu_v, bufm_v, sem_g, sem_wu, sem_wm):
        wid = lax.axis_index("s") * NC + lax.axis_index("c")
        base = wid * N_CHUNK
        pltpu.sync_copy(uid_hbm.at[pl.ds(base, N_CHUNK)], idxu_v)
        pltpu.sync_copy(mid_hbm.at[pl.ds(base, N_CHUNK)], idxm_v)
        gu = [pltpu.async_copy(zu_hbm.at[idxu_v.at[j]], bufu_v.at[j], sem_g)
              for j in range(N_CHUNK)]
        gm = [pltpu.async_copy(zm_hbm.at[idxm_v.at[j]], bufm_v.at[j], sem_g)
              for j in range(N_CHUNK - 1)]
        for c in gu:
            c.wait()
        wu = pltpu.async_copy(bufu_v, u_out.at[pl.ds(base, N_CHUNK)], sem_wu)
        for c in gm:
            c.wait()
        wm0 = pltpu.async_copy(
            bufm_v, m_out.at[pl.ds(base, N_CHUNK - 1)], sem_wm)
        wu.wait()
        glast = pltpu.async_copy(
            zm_hbm.at[idxm_v.at[N_CHUNK - 1]], bufu_v.at[0], sem_g)
        glast.wait()
        wm1 = pltpu.async_copy(
            bufu_v.at[0], m_out.at[base + N_CHUNK - 1], sem_wm)
        wm0.wait()
        wm1.wait()

    return k(uidx2d, midx2d, zu, zm)


BR = 2048


def _mlp_body(gu_ref, gm_ref, pu_ref, pm_ref, summ_ref, b1_ref, w2_ref,
              b2_ref, o_ref):
    grp = jax.lax.broadcasted_iota(jnp.int32, (BR, D2), 1) >> 3
    sel = (jnp.where(grp == pu_ref[...], gu_ref[...], 0.0)
           + jnp.where(grp == pm_ref[...], gm_ref[...], 0.0))
    x = jnp.dot(sel, summ_ref[...], preferred_element_type=jnp.float32)
    h = jnp.maximum(x + b1_ref[...], 0.0)
    o = jnp.dot(h, w2_ref[...], preferred_element_type=jnp.float32) + b2_ref[...]
    o_ref[...] = jax.nn.sigmoid(o) * 4.0 + 1.0


def _tc_mlp(gu, gm, pu, pm, summ, b1, W2, b2):
    grid = (BATCH // BR,)
    return pl.pallas_call(
        _mlp_body,
        grid=grid,
        in_specs=[
            pl.BlockSpec((BR, D2), lambda i: (i, 0)),
            pl.BlockSpec((BR, D2), lambda i: (i, 0)),
            pl.BlockSpec((BR, 1), lambda i: (i, 0)),
            pl.BlockSpec((BR, 1), lambda i: (i, 0)),
            pl.BlockSpec((D2, NH), lambda i: (0, 0)),
            pl.BlockSpec((1, NH), lambda i: (0, 0)),
            pl.BlockSpec((NH, 1), lambda i: (0, 0)),
            pl.BlockSpec((1, 1), lambda i: (0, 0)),
        ],
        out_specs=pl.BlockSpec((BR, 1), lambda i: (i, 0)),
        out_shape=jax.ShapeDtypeStruct((BATCH, 1), jnp.float32),
    )(gu, gm, pu, pm, summ, b1, W2, b2)


def kernel(user_ids, movie_ids, user_table, movie_table, W1, b1, W2, b2):
    uid = user_ids.astype(jnp.int32)
    mid = movie_ids.astype(jnp.int32)
    eye128 = jnp.eye(D2, dtype=jnp.float32)
    zu = _proj_stage(user_table.T, W1[:EMBED_DIM], eye128, NUM_USERS)
    zm = _proj_stage(movie_table.T, W1[EMBED_DIM:], eye128, NUM_MOVIES)
    urow = ((uid >> 14) << 10) | (uid & (SEGF - 1))
    mrow = ((mid >> 14) << 10) | (mid & (SEGF - 1))
    gu3, gm3 = _sc_gather(urow.reshape(NW * N_CHUNK, CHUNK),
                          mrow.reshape(NW * N_CHUNK, CHUNK), zu, zm)
    gu = gu3.reshape(BATCH, D2)
    gm = gm3.reshape(BATCH, D2)
    summ = jnp.tile(jnp.eye(NH, dtype=jnp.float32), (UPR, 1))
    out = _tc_mlp(gu, gm, ((uid >> 10) & 15).reshape(BATCH, 1),
                  ((mid >> 10) & 15).reshape(BATCH, 1), summ,
                  b1.reshape(1, NH), W2, b2.reshape(1, 1))
    return out.reshape(BATCH)


# speedup vs baseline: 4.1886x; 1.1005x over previous
import functools

import jax
import jax.numpy as jnp
from jax import lax
from jax.experimental import pallas as pl
from jax.experimental.pallas import tpu as pltpu
from jax.experimental.pallas import tpu_sc as plsc

NUM_USERS = 1000000
NUM_MOVIES = 100000
EMBED_DIM = 64
BATCH = 16384
NH = 8

NC = 2
NS = 16
NW = NC * NS
B_PER_W = BATCH // NW
N_CHUNK = 4
CHUNK = B_PER_W // N_CHUNK
D2 = 128
UPR = D2 // NH
SEGF = 2048
UPB = SEGF * UPR


def _proj_body(x_ref, w_ref, eye_ref, o_ref, *, n):
    dn = (((0,), (0,)), ((), ()))
    p = lax.dot_general(w_ref[...], x_ref[...], dn,
                        preferred_element_type=jnp.float32)
    nvalid = jnp.int32(n) - pl.program_id(0) * UPB
    lane = jax.lax.broadcasted_iota(jnp.int32, (NH, UPB), 1)
    p = jnp.where(lane < nvalid, p, 0.0)
    p2 = jnp.concatenate([p[:, g * SEGF:(g + 1) * SEGF] for g in range(UPR)],
                         axis=0)
    o_ref[...] = lax.dot_general(p2, eye_ref[...], dn,
                                 preferred_element_type=jnp.float32)


def _proj_stage(xt, w, eye128, n):
    nb = -(-n // UPB)
    return pl.pallas_call(
        functools.partial(_proj_body, n=n),
        grid=(nb,),
        in_specs=[
            pl.BlockSpec((EMBED_DIM, UPB), lambda b: (0, b)),
            pl.BlockSpec((EMBED_DIM, NH), lambda b: (0, 0)),
            pl.BlockSpec((D2, D2), lambda b: (0, 0)),
        ],
        out_specs=pl.BlockSpec((SEGF, D2), lambda b: (b, 0)),
        out_shape=jax.ShapeDtypeStruct((nb * SEGF, D2), jnp.float32),
        compiler_params=pltpu.CompilerParams(
            dimension_semantics=("parallel",)),
    )(xt, w, eye128)


def _sc_gather(uidx2d, midx2d, zu, zm):
    mesh = plsc.VectorSubcoreMesh(core_axis_name="c", subcore_axis_name="s")
    out_sds = jax.ShapeDtypeStruct((NW * N_CHUNK, CHUNK, D2), jnp.float32)

    @functools.partial(
        pl.kernel,
        out_type=(out_sds, out_sds),
        mesh=mesh,
        scratch_types=[
            pltpu.VMEM((N_CHUNK, CHUNK), jnp.int32),
            pltpu.VMEM((N_CHUNK, CHUNK), jnp.int32),
            pltpu.VMEM((N_CHUNK, CHUNK, D2), jnp.float32),
            pltpu.VMEM((N_CHUNK - 1, CHUNK, D2), jnp.float32),
            pltpu.SemaphoreType.DMA,
            pltpu.SemaphoreType.DMA,
            pltpu.SemaphoreType.DMA,
        ],
    )
    def k(uid_hbm, mid_hbm, zu_hbm, zm_hbm, u_out, m_out, idxu_v, idxm_v,
          bufu_v, bufm_v, sem_g, sem_wu, sem_wm):
        wid = lax.axis_index("s") * NC + lax.axis_index("c")
        base = wid * N_CHUNK
        pltpu.sync_copy(uid_hbm.at[pl.ds(base, N_CHUNK)], idxu_v)
        pltpu.sync_copy(mid_hbm.at[pl.ds(base, N_CHUNK)], idxm_v)
        gu = [pltpu.async_copy(zu_hbm.at[idxu_v.at[j]], bufu_v.at[j], sem_g)
              for j in range(N_CHUNK)]
        gm = [pltpu.async_copy(zm_hbm.at[idxm_v.at[j]], bufm_v.at[j], sem_g)
              for j in range(N_CHUNK - 1)]
        for c in gu:
            c.wait()
        wu = pltpu.async_copy(bufu_v, u_out.at[pl.ds(base, N_CHUNK)], sem_wu)
        for c in gm:
            c.wait()
        wm0 = pltpu.async_copy(
            bufm_v, m_out.at[pl.ds(base, N_CHUNK - 1)], sem_wm)
        wu.wait()
        glast = pltpu.async_copy(
            zm_hbm.at[idxm_v.at[N_CHUNK - 1]], bufu_v.at[0], sem_g)
        glast.wait()
        wm1 = pltpu.async_copy(
            bufu_v.at[0], m_out.at[base + N_CHUNK - 1], sem_wm)
        wm0.wait()
        wm1.wait()

    return k(uidx2d, midx2d, zu, zm)


BR = 2048


def _mlp_body(gu_ref, gm_ref, pu_ref, pm_ref, summ_ref, b1_ref, w2_ref,
              b2_ref, o_ref):
    grp = jax.lax.broadcasted_iota(jnp.int32, (BR, D2), 1) >> 3
    sel = (jnp.where(grp == pu_ref[...], gu_ref[...], 0.0)
           + jnp.where(grp == pm_ref[...], gm_ref[...], 0.0))
    x = jnp.dot(sel, summ_ref[...], preferred_element_type=jnp.float32)
    h = jnp.maximum(x + b1_ref[...], 0.0)
    o = jnp.dot(h, w2_ref[...], preferred_element_type=jnp.float32) + b2_ref[...]
    o_ref[...] = jax.nn.sigmoid(o) * 4.0 + 1.0


def _tc_mlp(gu, gm, pu, pm, summ, b1, W2, b2):
    grid = (BATCH // BR,)
    return pl.pallas_call(
        _mlp_body,
        grid=grid,
        in_specs=[
            pl.BlockSpec((BR, D2), lambda i: (i, 0)),
            pl.BlockSpec((BR, D2), lambda i: (i, 0)),
            pl.BlockSpec((BR, 1), lambda i: (i, 0)),
            pl.BlockSpec((BR, 1), lambda i: (i, 0)),
            pl.BlockSpec((D2, NH), lambda i: (0, 0)),
            pl.BlockSpec((1, NH), lambda i: (0, 0)),
            pl.BlockSpec((NH, 1), lambda i: (0, 0)),
            pl.BlockSpec((1, 1), lambda i: (0, 0)),
        ],
        out_specs=pl.BlockSpec((BR, 1), lambda i: (i, 0)),
        out_shape=jax.ShapeDtypeStruct((BATCH, 1), jnp.float32),
    )(gu, gm, pu, pm, summ, b1, W2, b2)


def kernel(user_ids, movie_ids, user_table, movie_table, W1, b1, W2, b2):
    uid = user_ids.astype(jnp.int32)
    mid = movie_ids.astype(jnp.int32)
    eye128 = jnp.eye(D2, dtype=jnp.float32)
    zu = _proj_stage(user_table.T, W1[:EMBED_DIM], eye128, NUM_USERS)
    zm = _proj_stage(movie_table.T, W1[EMBED_DIM:], eye128, NUM_MOVIES)
    urow = ((uid >> 15) << 11) | (uid & (SEGF - 1))
    mrow = ((mid >> 15) << 11) | (mid & (SEGF - 1))
    gu3, gm3 = _sc_gather(urow.reshape(NW * N_CHUNK, CHUNK),
                          mrow.reshape(NW * N_CHUNK, CHUNK), zu, zm)
    gu = gu3.reshape(BATCH, D2)
    gm = gm3.reshape(BATCH, D2)
    summ = jnp.tile(jnp.eye(NH, dtype=jnp.float32), (UPR, 1))
    out = _tc_mlp(gu, gm, ((uid >> 11) & 15).reshape(BATCH, 1),
                  ((mid >> 11) & 15).reshape(BATCH, 1), summ,
                  b1.reshape(1, NH), W2, b2.reshape(1, 1))
    return out.reshape(BATCH)


# speedup vs baseline: 4.2143x; 1.0062x over previous
import functools

import jax
import jax.numpy as jnp
from jax import lax
from jax.experimental import pallas as pl
from jax.experimental.pallas import tpu as pltpu
from jax.experimental.pallas import tpu_sc as plsc

NUM_USERS = 1000000
NUM_MOVIES = 100000
EMBED_DIM = 64
BATCH = 16384
NH = 8

NC = 2
NS = 16
NW = NC * NS
B_PER_W = BATCH // NW
N_CHUNK = 4
CHUNK = B_PER_W // N_CHUNK
D2 = 128
UPR = D2 // NH
SEGF = 4096
UPB = SEGF * UPR


def _proj_body(x_ref, w_ref, eye_ref, o_ref, *, n):
    dn = (((0,), (0,)), ((), ()))
    p = lax.dot_general(w_ref[...], x_ref[...], dn,
                        preferred_element_type=jnp.float32)
    nvalid = jnp.int32(n) - pl.program_id(0) * UPB
    lane = jax.lax.broadcasted_iota(jnp.int32, (NH, UPB), 1)
    p = jnp.where(lane < nvalid, p, 0.0)
    p2 = jnp.concatenate([p[:, g * SEGF:(g + 1) * SEGF] for g in range(UPR)],
                         axis=0)
    o_ref[...] = lax.dot_general(p2, eye_ref[...], dn,
                                 preferred_element_type=jnp.float32)


def _proj_stage(xt, w, eye128, n):
    nb = -(-n // UPB)
    return pl.pallas_call(
        functools.partial(_proj_body, n=n),
        grid=(nb,),
        in_specs=[
            pl.BlockSpec((EMBED_DIM, UPB), lambda b: (0, b)),
            pl.BlockSpec((EMBED_DIM, NH), lambda b: (0, 0)),
            pl.BlockSpec((D2, D2), lambda b: (0, 0)),
        ],
        out_specs=pl.BlockSpec((SEGF, D2), lambda b: (b, 0)),
        out_shape=jax.ShapeDtypeStruct((nb * SEGF, D2), jnp.float32),
        compiler_params=pltpu.CompilerParams(
            dimension_semantics=("parallel",)),
    )(xt, w, eye128)


def _sc_gather(uidx2d, midx2d, zu, zm):
    mesh = plsc.VectorSubcoreMesh(core_axis_name="c", subcore_axis_name="s")
    out_sds = jax.ShapeDtypeStruct((NW * N_CHUNK, CHUNK, D2), jnp.float32)

    @functools.partial(
        pl.kernel,
        out_type=(out_sds, out_sds),
        mesh=mesh,
        scratch_types=[
            pltpu.VMEM((N_CHUNK, CHUNK), jnp.int32),
            pltpu.VMEM((N_CHUNK, CHUNK), jnp.int32),
            pltpu.VMEM((N_CHUNK, CHUNK, D2), jnp.float32),
            pltpu.VMEM((N_CHUNK - 1, CHUNK, D2), jnp.float32),
            pltpu.SemaphoreType.DMA,
            pltpu.SemaphoreType.DMA,
            pltpu.SemaphoreType.DMA,
        ],
    )
    def k(uid_hbm, mid_hbm, zu_hbm, zm_hbm, u_out, m_out, idxu_v, idxm_v,
          bufu_v, bufm_v, sem_g, sem_wu, sem_wm):
        wid = lax.axis_index("s") * NC + lax.axis_index("c")
        base = wid * N_CHUNK
        pltpu.sync_copy(uid_hbm.at[pl.ds(base, N_CHUNK)], idxu_v)
        pltpu.sync_copy(mid_hbm.at[pl.ds(base, N_CHUNK)], idxm_v)
        gu = [pltpu.async_copy(zu_hbm.at[idxu_v.at[j]], bufu_v.at[j], sem_g)
              for j in range(N_CHUNK)]
        gm = [pltpu.async_copy(zm_hbm.at[idxm_v.at[j]], bufm_v.at[j], sem_g)
              for j in range(N_CHUNK - 1)]
        for c in gu:
            c.wait()
        wu = pltpu.async_copy(bufu_v, u_out.at[pl.ds(base, N_CHUNK)], sem_wu)
        for c in gm:
            c.wait()
        wm0 = pltpu.async_copy(
            bufm_v, m_out.at[pl.ds(base, N_CHUNK - 1)], sem_wm)
        wu.wait()
        glast = pltpu.async_copy(
            zm_hbm.at[idxm_v.at[N_CHUNK - 1]], bufu_v.at[0], sem_g)
        glast.wait()
        wm1 = pltpu.async_copy(
            bufu_v.at[0], m_out.at[base + N_CHUNK - 1], sem_wm)
        wm0.wait()
        wm1.wait()

    return k(uidx2d, midx2d, zu, zm)


BR = 2048


def _mlp_body(gu_ref, gm_ref, pu_ref, pm_ref, summ_ref, b1_ref, w2_ref,
              b2_ref, o_ref):
    grp = jax.lax.broadcasted_iota(jnp.int32, (BR, D2), 1) >> 3
    sel = (jnp.where(grp == pu_ref[...], gu_ref[...], 0.0)
           + jnp.where(grp == pm_ref[...], gm_ref[...], 0.0))
    x = jnp.dot(sel, summ_ref[...], preferred_element_type=jnp.float32)
    h = jnp.maximum(x + b1_ref[...], 0.0)
    o = jnp.dot(h, w2_ref[...], preferred_element_type=jnp.float32) + b2_ref[...]
    o_ref[...] = jax.nn.sigmoid(o) * 4.0 + 1.0


def _tc_mlp(gu, gm, pu, pm, summ, b1, W2, b2):
    grid = (BATCH // BR,)
    return pl.pallas_call(
        _mlp_body,
        grid=grid,
        in_specs=[
            pl.BlockSpec((BR, D2), lambda i: (i, 0)),
            pl.BlockSpec((BR, D2), lambda i: (i, 0)),
            pl.BlockSpec((BR, 1), lambda i: (i, 0)),
            pl.BlockSpec((BR, 1), lambda i: (i, 0)),
            pl.BlockSpec((D2, NH), lambda i: (0, 0)),
            pl.BlockSpec((1, NH), lambda i: (0, 0)),
            pl.BlockSpec((NH, 1), lambda i: (0, 0)),
            pl.BlockSpec((1, 1), lambda i: (0, 0)),
        ],
        out_specs=pl.BlockSpec((BR, 1), lambda i: (i, 0)),
        out_shape=jax.ShapeDtypeStruct((BATCH, 1), jnp.float32),
    )(gu, gm, pu, pm, summ, b1, W2, b2)


def kernel(user_ids, movie_ids, user_table, movie_table, W1, b1, W2, b2):
    uid = user_ids.astype(jnp.int32)
    mid = movie_ids.astype(jnp.int32)
    eye128 = jnp.eye(D2, dtype=jnp.float32)
    zu = _proj_stage(user_table.T, W1[:EMBED_DIM], eye128, NUM_USERS)
    zm = _proj_stage(movie_table.T, W1[EMBED_DIM:], eye128, NUM_MOVIES)
    urow = ((uid >> 16) << 12) | (uid & (SEGF - 1))
    mrow = ((mid >> 16) << 12) | (mid & (SEGF - 1))
    gu3, gm3 = _sc_gather(urow.reshape(NW * N_CHUNK, CHUNK),
                          mrow.reshape(NW * N_CHUNK, CHUNK), zu, zm)
    gu = gu3.reshape(BATCH, D2)
    gm = gm3.reshape(BATCH, D2)
    summ = jnp.tile(jnp.eye(NH, dtype=jnp.float32), (UPR, 1))
    out = _tc_mlp(gu, gm, ((uid >> 12) & 15).reshape(BATCH, 1),
                  ((mid >> 12) & 15).reshape(BATCH, 1), summ,
                  b1.reshape(1, NH), W2, b2.reshape(1, 1))
    return out.reshape(BATCH)
